# Initial kernel scaffold; baseline (speedup 1.0000x reference)
import jax
import jax.numpy as jnp
from jax.experimental import pallas as pl


def kernel(x, edge_attr, edge_weight, edge_dst, edge_src):
    raise NotImplementedError("write your pallas kernel here")


# speedup vs baseline: 2.5028x; 2.5028x over previous
import functools

import jax
import jax.numpy as jnp
from jax import lax
from jax.experimental import pallas as pl
from jax.experimental.pallas import tpu as pltpu
from jax.experimental.pallas import tpu_sc as plsc

MUL = 16
LANES = 16
NT = 16
B = 512
CHUNK = 128
NCH = B // CHUNK

_GDN = lax.GatherDimensionNumbers(
    offset_dims=(), collapsed_slice_dims=(0,), start_index_map=(0,))


def _lane_bcast(vec, j):
    idx = jnp.full((LANES, 1), j, dtype=jnp.int32)
    return lax.gather(vec, idx, _GDN, (1,),
                      mode=lax.GatherScatterMode.PROMISE_IN_BOUNDS)


def _make_sc_kernel(Npad, E0, Ep, K):
    CH_D = Npad // NT
    NZ = CH_D // CHUNK
    ZT = CH_D % CHUNK
    mesh = plsc.VectorSubcoreMesh(core_axis_name="c", subcore_axis_name="s")

    @functools.partial(
        pl.kernel, mesh=mesh,
        compiler_params=pltpu.CompilerParams(use_tc_tiling_on_sc=False),
        out_type=jax.ShapeDtypeStruct((4 * Npad, MUL), jnp.float32),
        scratch_types=[
            pltpu.VMEM((NCH, CHUNK), jnp.int32),
            pltpu.VMEM((NCH, CHUNK), jnp.int32),
            pltpu.VMEM((B,), jnp.float32),
            pltpu.VMEM((B, MUL), jnp.float32),
            pltpu.VMEM((B, MUL), jnp.float32),
            pltpu.VMEM((CHUNK, MUL), jnp.float32),
            pltpu.VMEM_SHARED((Npad, MUL), jnp.float32),
            pltpu.VMEM_SHARED((Npad, MUL), jnp.float32),
            pltpu.SemaphoreType.DMA,
        ])
    def sc_kernel(xqh, attrh, wh, srch, dsth, outh,
                  src_v, dst_v, attr_v, w_v, xr_v, zero_v, xsp, acc, sem):
        cid = lax.axis_index("c")
        sid = lax.axis_index("s")

        def zrow(r, c):
            zero_v[r, pl.ds(0, LANES)] = jnp.zeros((LANES,), jnp.float32)
            return c
        lax.fori_loop(0, CHUNK, zrow, 0)

        def sweep(qrow0, wrow0):
            r0 = sid * CH_D
            for t in range(NZ):
                pltpu.sync_copy(zero_v, acc.at[pl.ds(r0 + t * CHUNK, CHUNK)])
            if ZT:
                pltpu.sync_copy(zero_v.at[pl.ds(0, ZT)],
                                acc.at[pl.ds(r0 + NZ * CHUNK, ZT)])
            pltpu.sync_copy(xqh.at[pl.ds(qrow0 + r0, CH_D)],
                            xsp.at[pl.ds(r0, CH_D)])
            plsc.subcore_barrier()

            def block(kb, c):
                blk = sid * K + kb
                ebase = blk * B
                rb = blk * NCH
                pltpu.sync_copy(srch.at[pl.ds(rb, NCH)], src_v)
                pltpu.sync_copy(dsth.at[pl.ds(rb, NCH)], dst_v)
                pltpu.sync_copy(attrh.at[pl.ds(ebase, B)], attr_v)
                pltpu.sync_copy(wh.at[pl.ds(wrow0 + ebase, B)], w_v)
                cps = [pltpu.async_copy(xsp.at[src_v.at[j]],
                                        xr_v.at[pl.ds(j * CHUNK, CHUNK)], sem)
                       for j in range(NCH)]
                for cp in cps:
                    cp.wait()

                def group(g, gc):
                    s16 = attr_v[pl.ds(g * LANES, LANES)]
                    for j2 in range(LANES):
                        e = g * LANES + j2
                        sj = _lane_bcast(s16, j2)
                        wv = w_v[e, pl.ds(0, MUL)]
                        xr_v[e, pl.ds(0, MUL)] = (
                            xr_v[e, pl.ds(0, MUL)] * (sj * wv))
                    return gc
                lax.fori_loop(0, B // LANES, group, 0)

                for j in range(NCH):
                    pltpu.sync_copy(xr_v.at[pl.ds(j * CHUNK, CHUNK)],
                                    acc.at[dst_v.at[j]], add=True)
                return c
            lax.fori_loop(0, K, block, 0)

            plsc.subcore_barrier()
            pltpu.sync_copy(acc.at[pl.ds(r0, CH_D)],
                            outh.at[pl.ds(qrow0 + r0, CH_D)])

        @pl.when(cid == 0)
        def _():
            sweep(0 * Npad, 0)
            sweep(1 * Npad, Ep)

        @pl.when(cid == 1)
        def _():
            sweep(2 * Npad, Ep)
            sweep(3 * Npad, Ep)

    return sc_kernel


def kernel(x, edge_attr, edge_weight, edge_dst, edge_src):
    N, F = x.shape
    E0 = edge_src.shape[0]
    K = -(-E0 // (NT * B))
    Ep = NT * K * B
    Npad = -(-N // 128) * 128

    x = x.astype(jnp.float32)
    xc = x[:, MUL:].reshape(N, MUL, 3)
    rowpad = ((0, Npad - N), (0, 0))
    xq = jnp.concatenate(
        [jnp.pad(x[:, :MUL], rowpad),
         jnp.pad(xc[:, :, 0], rowpad),
         jnp.pad(xc[:, :, 1], rowpad),
         jnp.pad(xc[:, :, 2], rowpad)], axis=0)

    pad = Ep - E0
    srcp = jnp.concatenate(
        [edge_src.astype(jnp.int32), jnp.zeros((pad,), jnp.int32)]
    ).reshape(Ep // CHUNK, CHUNK)
    dstp = jnp.concatenate(
        [edge_dst.astype(jnp.int32), jnp.full((pad,), N, jnp.int32)]
    ).reshape(Ep // CHUNK, CHUNK)
    attrp = jnp.concatenate(
        [edge_attr.reshape(E0).astype(jnp.float32),
         jnp.zeros((pad,), jnp.float32)])
    wf = edge_weight.astype(jnp.float32)
    epad = ((0, pad), (0, 0))
    wq = jnp.concatenate(
        [jnp.pad(wf[:, :MUL], epad),
         jnp.pad(wf[:, MUL:], epad)], axis=0)

    sc = _make_sc_kernel(Npad, E0, Ep, K)
    op = sc(xq, attrp, wq, srcp, dstp)

    o = op.reshape(4, Npad, MUL)[:, :N, :]
    out1 = jnp.stack([o[1], o[2], o[3]], axis=-1).reshape(N, 3 * MUL)
    return jnp.concatenate([o[0], out1], axis=1)


# speedup vs baseline: 2.6746x; 1.0687x over previous
import functools

import jax
import jax.numpy as jnp
from jax import lax
from jax.experimental import pallas as pl
from jax.experimental.pallas import tpu as pltpu
from jax.experimental.pallas import tpu_sc as plsc

MUL = 16
LANES = 16
W2 = 2 * MUL
NT = 16
B = 256
CHUNK = 128
NCH = B // CHUNK

_GDN = lax.GatherDimensionNumbers(
    offset_dims=(), collapsed_slice_dims=(0,), start_index_map=(0,))


def _lane_bcast(vec, j):
    idx = jnp.full((LANES, 1), j, dtype=jnp.int32)
    return lax.gather(vec, idx, _GDN, (1,),
                      mode=lax.GatherScatterMode.PROMISE_IN_BOUNDS)


def _make_sc_kernel(Npad, Ep, K):
    CH_D = Npad // NT
    NZ = CH_D // CHUNK
    ZT = CH_D % CHUNK
    mesh = plsc.VectorSubcoreMesh(core_axis_name="c", subcore_axis_name="s")

    @functools.partial(
        pl.kernel, mesh=mesh,
        compiler_params=pltpu.CompilerParams(use_tc_tiling_on_sc=False),
        out_type=jax.ShapeDtypeStruct((2 * Npad, W2), jnp.float32),
        scratch_types=[
            pltpu.VMEM((NCH, CHUNK), jnp.int32),
            pltpu.VMEM((NCH, CHUNK), jnp.int32),
            pltpu.VMEM((B,), jnp.float32),
            pltpu.VMEM((B, W2), jnp.float32),
            pltpu.VMEM((B, W2), jnp.float32),
            pltpu.VMEM((CHUNK, W2), jnp.float32),
            pltpu.VMEM_SHARED((Npad, W2), jnp.float32),
            pltpu.SemaphoreType.DMA,
        ])
    def sc_kernel(x0h, x1h, attrh, wh, srch, dsth, outh,
                  src_v, dst_v, attr_v, w_v, xr_v, zero_v, acc, sem):
        cid = lax.axis_index("c")
        sid = lax.axis_index("s")

        def zrow(r, c):
            z = jnp.zeros((LANES,), jnp.float32)
            zero_v[r, pl.ds(0, LANES)] = z
            zero_v[r, pl.ds(LANES, LANES)] = z
            return c
        lax.fori_loop(0, CHUNK, zrow, 0)

        r0 = sid * CH_D
        for t in range(NZ):
            pltpu.sync_copy(zero_v, acc.at[pl.ds(r0 + t * CHUNK, CHUNK)])
        if ZT:
            pltpu.sync_copy(zero_v.at[pl.ds(0, ZT)],
                            acc.at[pl.ds(r0 + NZ * CHUNK, ZT)])
        plsc.subcore_barrier()

        def blocks(xh, wa, wb):
            def block(kb, c):
                blk = sid * K + kb
                ebase = blk * B
                rb = blk * NCH
                pltpu.sync_copy(srch.at[pl.ds(rb, NCH)], src_v)
                pltpu.sync_copy(dsth.at[pl.ds(rb, NCH)], dst_v)
                pltpu.sync_copy(attrh.at[pl.ds(ebase, B)], attr_v)
                pltpu.sync_copy(wh.at[pl.ds(ebase, B)], w_v)
                cps = [pltpu.async_copy(xh.at[src_v.at[j]],
                                        xr_v.at[pl.ds(j * CHUNK, CHUNK)], sem)
                       for j in range(NCH)]
                for cp in cps:
                    cp.wait()

                def group(g, gc):
                    s16 = attr_v[pl.ds(g * LANES, LANES)]
                    for j2 in range(LANES):
                        e = g * LANES + j2
                        sj = _lane_bcast(s16, j2)
                        wva = w_v[e, pl.ds(wa, MUL)]
                        wvb = w_v[e, pl.ds(wb, MUL)]
                        xr_v[e, pl.ds(0, MUL)] = (
                            xr_v[e, pl.ds(0, MUL)] * (sj * wva))
                        xr_v[e, pl.ds(MUL, MUL)] = (
                            xr_v[e, pl.ds(MUL, MUL)] * (sj * wvb))
                    return gc
                lax.fori_loop(0, B // LANES, group, 0)

                for j in range(NCH):
                    pltpu.sync_copy(xr_v.at[pl.ds(j * CHUNK, CHUNK)],
                                    acc.at[dst_v.at[j]], add=True)
                return c
            lax.fori_loop(0, K, block, 0)

        @pl.when(cid == 0)
        def _():
            blocks(x0h, 0, MUL)

        @pl.when(cid == 1)
        def _():
            blocks(x1h, MUL, MUL)

        plsc.subcore_barrier()
        pltpu.sync_copy(acc.at[pl.ds(r0, CH_D)],
                        outh.at[pl.ds(cid * Npad + r0, CH_D)])

    return sc_kernel


def kernel(x, edge_attr, edge_weight, edge_dst, edge_src):
    N, F = x.shape
    E0 = edge_src.shape[0]
    K = -(-E0 // (NT * B))
    Ep = NT * K * B
    Npad = -(-N // 128) * 128

    x = x.astype(jnp.float32)
    xc = x[:, MUL:].reshape(N, MUL, 3)
    rowpad = ((0, Npad - N), (0, 0))
    x0p = jnp.pad(jnp.concatenate([x[:, :MUL], xc[:, :, 0]], axis=1), rowpad)
    x1p = jnp.pad(jnp.concatenate([xc[:, :, 1], xc[:, :, 2]], axis=1), rowpad)

    pad = Ep - E0
    srcp = jnp.concatenate(
        [edge_src.astype(jnp.int32), jnp.zeros((pad,), jnp.int32)]
    ).reshape(Ep // CHUNK, CHUNK)
    dstp = jnp.concatenate(
        [edge_dst.astype(jnp.int32), jnp.full((pad,), N, jnp.int32)]
    ).reshape(Ep // CHUNK, CHUNK)
    attrp = jnp.concatenate(
        [edge_attr.reshape(E0).astype(jnp.float32),
         jnp.zeros((pad,), jnp.float32)])
    wq = jnp.pad(edge_weight.astype(jnp.float32), ((0, pad), (0, 0)))

    sc = _make_sc_kernel(Npad, Ep, K)
    op = sc(x0p, x1p, attrp, wq, srcp, dstp)

    o = op.reshape(2, Npad, W2)[:, :N, :]
    out1 = jnp.stack([o[0][:, MUL:], o[1][:, :MUL], o[1][:, MUL:]],
                     axis=-1).reshape(N, 3 * MUL)
    return jnp.concatenate([o[0][:, :MUL], out1], axis=1)


# speedup vs baseline: 2.7893x; 1.0429x over previous
import functools

import jax
import jax.numpy as jnp
from jax import lax
from jax.experimental import pallas as pl
from jax.experimental.pallas import tpu as pltpu
from jax.experimental.pallas import tpu_sc as plsc

MUL = 16
LANES = 16
W2 = 2 * MUL
NT = 16
B = 256
CHUNK = 128
NCH = B // CHUNK
ZR = 32


def _make_sc_kernel(Npad, Ep, K):
    CH_D = Npad // NT
    NZ = CH_D // ZR
    ZT = CH_D % ZR
    mesh = plsc.VectorSubcoreMesh(core_axis_name="c", subcore_axis_name="s")

    @functools.partial(
        pl.kernel, mesh=mesh,
        compiler_params=pltpu.CompilerParams(use_tc_tiling_on_sc=False),
        out_type=jax.ShapeDtypeStruct((2 * Npad, W2), jnp.float32),
        scratch_types=[
            pltpu.VMEM((NCH, CHUNK), jnp.int32),
            pltpu.VMEM((NCH, CHUNK), jnp.int32),
            pltpu.VMEM((NCH, CHUNK), jnp.int32),
            pltpu.VMEM((B,), jnp.float32),
            pltpu.VMEM((B, W2), jnp.float32),
            pltpu.VMEM((B, W2), jnp.float32),
            pltpu.VMEM((B, W2), jnp.float32),
            pltpu.VMEM((ZR, W2), jnp.float32),
            pltpu.VMEM_SHARED((Npad, W2), jnp.float32),
            pltpu.SemaphoreType.DMA,
            pltpu.SemaphoreType.DMA,
            pltpu.SemaphoreType.DMA,
        ])
    def sc_kernel(x0h, x1h, attrh, wh, srch, dsth, outh,
                  src_v, dst0_v, dst1_v, attr_v, w_v, xr0_v, xr1_v, zero_v,
                  acc, gsem, ssem0, ssem1):
        cid = lax.axis_index("c")
        sid = lax.axis_index("s")
        dst_v = (dst0_v, dst1_v)
        xr_v = (xr0_v, xr1_v)
        ssem = (ssem0, ssem1)

        def zrow(r, c):
            z = jnp.zeros((LANES,), jnp.float32)
            zero_v[r, pl.ds(0, LANES)] = z
            zero_v[r, pl.ds(LANES, LANES)] = z
            return c
        lax.fori_loop(0, ZR, zrow, 0)

        r0 = sid * CH_D
        for t in range(NZ):
            pltpu.sync_copy(zero_v, acc.at[pl.ds(r0 + t * ZR, ZR)])
        if ZT:
            pltpu.sync_copy(zero_v.at[pl.ds(0, ZT)],
                            acc.at[pl.ds(r0 + NZ * ZR, ZT)])
        plsc.subcore_barrier()

        def scat_waits(p):
            for j in range(NCH):
                pltpu.make_async_copy(
                    xr_v[p].at[pl.ds(j * CHUNK, CHUNK)],
                    acc.at[dst_v[p].at[j]], ssem[p]).wait()

        def blocks(xh, wa, wb):
            def phase(i, p, blk):
                ebase = blk * B
                rb = blk * NCH

                @pl.when(i > 0)
                def _():
                    scat_waits(p)

                pltpu.sync_copy(srch.at[pl.ds(rb, NCH)], src_v)
                pltpu.sync_copy(dsth.at[pl.ds(rb, NCH)], dst_v[p])
                pltpu.sync_copy(attrh.at[pl.ds(ebase, B)], attr_v)
                pltpu.sync_copy(wh.at[pl.ds(ebase, B)], w_v)
                cps = [pltpu.async_copy(
                           xh.at[src_v.at[j]],
                           xr_v[p].at[pl.ds(j * CHUNK, CHUNK)], gsem)
                       for j in range(NCH)]
                for cp in cps:
                    cp.wait()

                @plsc.parallel_loop(0, B // LANES)
                def group(g):
                    s16 = attr_v[pl.ds(g * LANES, LANES)]
                    for j2 in range(LANES):
                        e = g * LANES + j2
                        sj = jnp.broadcast_to(s16[j2], (LANES,))
                        swa = sj * w_v[e, pl.ds(wa, MUL)]
                        if wb == wa:
                            swb = swa
                        else:
                            swb = sj * w_v[e, pl.ds(wb, MUL)]
                        xr_v[p][e, pl.ds(0, MUL)] = (
                            xr_v[p][e, pl.ds(0, MUL)] * swa)
                        xr_v[p][e, pl.ds(MUL, MUL)] = (
                            xr_v[p][e, pl.ds(MUL, MUL)] * swb)

                for j in range(NCH):
                    pltpu.async_copy(xr_v[p].at[pl.ds(j * CHUNK, CHUNK)],
                                     acc.at[dst_v[p].at[j]], ssem[p],
                                     add=True)

            def pair(i, c):
                blk = sid * K + 2 * i
                phase(i, 0, blk)
                phase(i, 1, blk + 1)
                return c
            lax.fori_loop(0, K // 2, pair, 0)
            scat_waits(0)
            scat_waits(1)

        @pl.when(cid == 0)
        def _():
            blocks(x0h, 0, MUL)

        @pl.when(cid == 1)
        def _():
            blocks(x1h, MUL, MUL)

        plsc.subcore_barrier()
        pltpu.sync_copy(acc.at[pl.ds(r0, CH_D)],
                        outh.at[pl.ds(cid * Npad + r0, CH_D)])

    return sc_kernel


def kernel(x, edge_attr, edge_weight, edge_dst, edge_src):
    N, F = x.shape
    E0 = edge_src.shape[0]
    K = -(-E0 // (NT * B))
    K = K + (K % 2)
    Ep = NT * K * B
    Npad = -(-N // 128) * 128

    x = x.astype(jnp.float32)
    xc = x[:, MUL:].reshape(N, MUL, 3)
    rowpad = ((0, Npad - N), (0, 0))
    x0p = jnp.pad(jnp.concatenate([x[:, :MUL], xc[:, :, 0]], axis=1), rowpad)
    x1p = jnp.pad(jnp.concatenate([xc[:, :, 1], xc[:, :, 2]], axis=1), rowpad)

    pad = Ep - E0
    srcp = jnp.concatenate(
        [edge_src.astype(jnp.int32), jnp.zeros((pad,), jnp.int32)]
    ).reshape(Ep // CHUNK, CHUNK)
    dstp = jnp.concatenate(
        [edge_dst.astype(jnp.int32), jnp.full((pad,), N, jnp.int32)]
    ).reshape(Ep // CHUNK, CHUNK)
    attrp = jnp.concatenate(
        [edge_attr.reshape(E0).astype(jnp.float32),
         jnp.zeros((pad,), jnp.float32)])
    wq = jnp.pad(edge_weight.astype(jnp.float32), ((0, pad), (0, 0)))

    sc = _make_sc_kernel(Npad, Ep, K)
    op = sc(x0p, x1p, attrp, wq, srcp, dstp)

    o = op.reshape(2, Npad, W2)[:, :N, :]
    out1 = jnp.stack([o[0][:, MUL:], o[1][:, :MUL], o[1][:, MUL:]],
                     axis=-1).reshape(N, 3 * MUL)
    return jnp.concatenate([o[0][:, :MUL], out1], axis=1)


# speedup vs baseline: 3.1967x; 1.1461x over previous
import functools

import jax
import jax.numpy as jnp
from jax import lax
from jax.experimental import pallas as pl
from jax.experimental.pallas import tpu as pltpu
from jax.experimental.pallas import tpu_sc as plsc

MUL = 16
LANES = 16
W2 = 2 * MUL
NT = 16
B = 256
CHUNK = 128
NCH = B // CHUNK
ZR = 32


def _make_sc_kernel(Npad, E0, Ep, K):
    CH_D = Npad // NT
    RBV = E0 // CHUNK - NCH
    WBV = E0 - B
    NZ = CH_D // ZR
    ZT = CH_D % ZR
    mesh = plsc.VectorSubcoreMesh(core_axis_name="c", subcore_axis_name="s")

    @functools.partial(
        pl.kernel, mesh=mesh,
        compiler_params=pltpu.CompilerParams(use_tc_tiling_on_sc=False),
        out_type=(jax.ShapeDtypeStruct((Npad, W2), jnp.float32),
                  jax.ShapeDtypeStruct((Npad, W2), jnp.float32)),
        scratch_types=[
            pltpu.VMEM((NCH, CHUNK), jnp.int32),
            pltpu.VMEM((NCH, CHUNK), jnp.int32),
            pltpu.VMEM((NCH, CHUNK), jnp.int32),
            pltpu.VMEM((B,), jnp.float32),
            pltpu.VMEM((B, W2), jnp.float32),
            pltpu.VMEM((B, W2), jnp.float32),
            pltpu.VMEM((B, W2), jnp.float32),
            pltpu.VMEM((ZR, W2), jnp.float32),
            pltpu.VMEM_SHARED((Npad, W2), jnp.float32),
            pltpu.SemaphoreType.DMA,
            pltpu.SemaphoreType.DMA,
            pltpu.SemaphoreType.DMA,
        ])
    def sc_kernel(x0h, x1h, attrh, wh, srch, dsth, out0h, out1h,
                  src_v, dst0_v, dst1_v, attr_v, w_v, xr0_v, xr1_v, zero_v,
                  acc, gsem, ssem0, ssem1):
        cid = lax.axis_index("c")
        sid = lax.axis_index("s")
        dst_v = (dst0_v, dst1_v)
        xr_v = (xr0_v, xr1_v)
        ssem = (ssem0, ssem1)

        def zrow(r, c):
            z = jnp.zeros((LANES,), jnp.float32)
            zero_v[r, pl.ds(0, LANES)] = z
            zero_v[r, pl.ds(LANES, LANES)] = z
            return c
        lax.fori_loop(0, ZR, zrow, 0)

        r0 = sid * CH_D
        for t in range(NZ):
            pltpu.sync_copy(zero_v, acc.at[pl.ds(r0 + t * ZR, ZR)])
        if ZT:
            pltpu.sync_copy(zero_v.at[pl.ds(0, ZT)],
                            acc.at[pl.ds(r0 + NZ * ZR, ZT)])
        plsc.subcore_barrier()

        def scat_waits(p):
            for j in range(NCH):
                pltpu.make_async_copy(
                    xr_v[p].at[pl.ds(j * CHUNK, CHUNK)],
                    acc.at[dst_v[p].at[j]], ssem[p]).wait()

        def blocks(xh, wa, wb):
            def phase(i, p, blk):
                ebase = blk * B
                rb = blk * NCH

                @pl.when(i > 0)
                def _():
                    scat_waits(p)

                rb_s = jnp.minimum(rb, RBV)
                web = pl.multiple_of(jnp.minimum(ebase, WBV), CHUNK)
                pltpu.sync_copy(srch.at[pl.ds(rb_s, NCH)], src_v)
                pltpu.sync_copy(dsth.at[pl.ds(rb, NCH)], dst_v[p])
                pltpu.sync_copy(attrh.at[pl.ds(ebase, B)], attr_v)
                pltpu.sync_copy(wh.at[pl.ds(web, B)], w_v)
                cps = [pltpu.async_copy(
                           xh.at[src_v.at[j]],
                           xr_v[p].at[pl.ds(j * CHUNK, CHUNK)], gsem)
                       for j in range(NCH)]
                for cp in cps:
                    cp.wait()

                @plsc.parallel_loop(0, B // LANES)
                def group(g):
                    s16 = attr_v[pl.ds(g * LANES, LANES)]
                    for j2 in range(LANES):
                        e = g * LANES + j2
                        sj = jnp.broadcast_to(s16[j2], (LANES,))
                        swa = sj * w_v[e, pl.ds(wa, MUL)]
                        if wb == wa:
                            swb = swa
                        else:
                            swb = sj * w_v[e, pl.ds(wb, MUL)]
                        xr_v[p][e, pl.ds(0, MUL)] = (
                            xr_v[p][e, pl.ds(0, MUL)] * swa)
                        xr_v[p][e, pl.ds(MUL, MUL)] = (
                            xr_v[p][e, pl.ds(MUL, MUL)] * swb)

                for j in range(NCH):
                    pltpu.async_copy(xr_v[p].at[pl.ds(j * CHUNK, CHUNK)],
                                     acc.at[dst_v[p].at[j]], ssem[p],
                                     add=True)

            def pair(i, c):
                blk = sid * K + 2 * i
                phase(i, 0, blk)
                phase(i, 1, blk + 1)
                return c
            lax.fori_loop(0, K // 2, pair, 0)
            scat_waits(0)
            scat_waits(1)

        @pl.when(cid == 0)
        def _():
            blocks(x0h, 0, MUL)
            plsc.subcore_barrier()
            pltpu.sync_copy(acc.at[pl.ds(r0, CH_D)],
                            out0h.at[pl.ds(r0, CH_D)])

        @pl.when(cid == 1)
        def _():
            blocks(x1h, MUL, MUL)
            plsc.subcore_barrier()
            pltpu.sync_copy(acc.at[pl.ds(r0, CH_D)],
                            out1h.at[pl.ds(r0, CH_D)])

    return sc_kernel


def kernel(x, edge_attr, edge_weight, edge_dst, edge_src):
    N, F = x.shape
    E0 = edge_src.shape[0]
    K = -(-E0 // (NT * B))
    K = K + (K % 2)
    Ep = NT * K * B
    Npad = -(-N // 128) * 128

    x = x.astype(jnp.float32)
    xc = x[:, MUL:].reshape(N, MUL, 3)
    rowpad = ((0, Npad - N), (0, 0))
    x0p = jnp.pad(jnp.concatenate([x[:, :MUL], xc[:, :, 0]], axis=1), rowpad)
    x1p = jnp.pad(jnp.concatenate([xc[:, :, 1], xc[:, :, 2]], axis=1), rowpad)

    pad = Ep - E0
    assert E0 % CHUNK == 0
    srcp = edge_src.astype(jnp.int32).reshape(E0 // CHUNK, CHUNK)
    dstp = jnp.concatenate(
        [edge_dst.astype(jnp.int32), jnp.full((pad,), N, jnp.int32)]
    ).reshape(Ep // CHUNK, CHUNK)
    attrp = jnp.concatenate(
        [edge_attr.reshape(E0).astype(jnp.float32),
         jnp.zeros((pad,), jnp.float32)])
    wq = edge_weight.astype(jnp.float32)

    sc = _make_sc_kernel(Npad, E0, Ep, K)
    op0, op1 = sc(x0p, x1p, attrp, wq, srcp, dstp)

    o = (op0[:N], op1[:N])
    out1 = jnp.stack([o[0][:, MUL:], o[1][:, :MUL], o[1][:, MUL:]],
                     axis=-1).reshape(N, 3 * MUL)
    return jnp.concatenate([o[0][:, :MUL], out1], axis=1)


# speedup vs baseline: 3.5290x; 1.1039x over previous
import functools

import jax
import jax.numpy as jnp
import numpy as np
from jax import lax
from jax.experimental import pallas as pl
from jax.experimental.pallas import tpu as pltpu
from jax.experimental.pallas import tpu_sc as plsc

MUL = 16
LANES = 16
W2 = 2 * MUL
NT = 16
B = 256
CHUNK = 128
NCH = B // CHUNK
ZR = 32


def _make_sc_kernel(Npad, E0, Ep, K):
    CH_D = Npad // NT
    RBV = E0 // CHUNK - NCH
    WBV = E0 - B
    NZ = CH_D // ZR
    ZT = CH_D % ZR
    mesh = plsc.VectorSubcoreMesh(core_axis_name="c", subcore_axis_name="s")

    @functools.partial(
        pl.kernel, mesh=mesh,
        compiler_params=pltpu.CompilerParams(use_tc_tiling_on_sc=False),
        out_type=(jax.ShapeDtypeStruct((Npad, W2), jnp.float32),
                  jax.ShapeDtypeStruct((Npad, W2), jnp.float32)),
        scratch_types=[
            pltpu.VMEM((NCH, CHUNK), jnp.int32),
            pltpu.VMEM((NCH, CHUNK), jnp.int32),
            pltpu.VMEM((NCH, CHUNK), jnp.int32),
            pltpu.VMEM((B,), jnp.float32),
            pltpu.VMEM((B, W2), jnp.float32),
            pltpu.VMEM((B, W2), jnp.float32),
            pltpu.VMEM((B, W2), jnp.float32),
            pltpu.VMEM((ZR, W2), jnp.float32),
            pltpu.VMEM_SHARED((Npad, W2), jnp.float32),
            pltpu.SemaphoreType.DMA,
            pltpu.SemaphoreType.DMA,
            pltpu.SemaphoreType.DMA,
        ])
    def sc_kernel(x0h, x1h, attrh, wh, srch, dsth, out0h, out1h,
                  src_v, dst0_v, dst1_v, attr_v, w_v, xr0_v, xr1_v, zero_v,
                  acc, gsem, ssem0, ssem1):
        cid = lax.axis_index("c")
        sid = lax.axis_index("s")
        dst_v = (dst0_v, dst1_v)
        xr_v = (xr0_v, xr1_v)
        ssem = (ssem0, ssem1)

        def zrow(r, c):
            z = jnp.zeros((LANES,), jnp.float32)
            zero_v[r, pl.ds(0, LANES)] = z
            zero_v[r, pl.ds(LANES, LANES)] = z
            return c
        lax.fori_loop(0, ZR, zrow, 0)

        r0 = sid * CH_D
        for t in range(NZ):
            pltpu.sync_copy(zero_v, acc.at[pl.ds(r0 + t * ZR, ZR)])
        if ZT:
            pltpu.sync_copy(zero_v.at[pl.ds(0, ZT)],
                            acc.at[pl.ds(r0 + NZ * ZR, ZT)])
        plsc.subcore_barrier()

        def scat_waits(p):
            for j in range(NCH):
                pltpu.make_async_copy(
                    xr_v[p].at[pl.ds(j * CHUNK, CHUNK)],
                    acc.at[dst_v[p].at[j]], ssem[p]).wait()

        def blocks(xh, wa, wb):
            def phase(i, p, blk):
                ebase = blk * B
                rb = blk * NCH

                @pl.when(i > 0)
                def _():
                    scat_waits(p)

                rb_s = jnp.minimum(rb, RBV)
                web = pl.multiple_of(jnp.minimum(ebase, WBV), CHUNK)
                pltpu.sync_copy(srch.at[pl.ds(rb_s, NCH)], src_v)
                pltpu.sync_copy(dsth.at[pl.ds(rb, NCH)], dst_v[p])
                pltpu.sync_copy(attrh.at[pl.ds(ebase, B)], attr_v)
                pltpu.sync_copy(wh.at[pl.ds(web, B)], w_v)
                cps = [pltpu.async_copy(
                           xh.at[src_v.at[j]],
                           xr_v[p].at[pl.ds(j * CHUNK, CHUNK)], gsem)
                       for j in range(NCH)]
                for cp in cps:
                    cp.wait()

                @plsc.parallel_loop(0, B // LANES)
                def group(g):
                    s16 = attr_v[pl.ds(g * LANES, LANES)]
                    for j2 in range(LANES):
                        e = g * LANES + j2
                        sj = jnp.broadcast_to(s16[j2], (LANES,))
                        swa = sj * w_v[e, pl.ds(wa, MUL)]
                        if wb == wa:
                            swb = swa
                        else:
                            swb = sj * w_v[e, pl.ds(wb, MUL)]
                        xr_v[p][e, pl.ds(0, MUL)] = (
                            xr_v[p][e, pl.ds(0, MUL)] * swa)
                        xr_v[p][e, pl.ds(MUL, MUL)] = (
                            xr_v[p][e, pl.ds(MUL, MUL)] * swb)

                for j in range(NCH):
                    pltpu.async_copy(xr_v[p].at[pl.ds(j * CHUNK, CHUNK)],
                                     acc.at[dst_v[p].at[j]], ssem[p],
                                     add=True)

            def pair(i, c):
                blk = sid * K + 2 * i
                phase(i, 0, blk)
                phase(i, 1, blk + 1)
                return c
            lax.fori_loop(0, K // 2, pair, 0)
            scat_waits(0)
            scat_waits(1)

        @pl.when(cid == 0)
        def _():
            blocks(x0h, 0, MUL)
            plsc.subcore_barrier()
            pltpu.sync_copy(acc.at[pl.ds(r0, CH_D)],
                            out0h.at[pl.ds(r0, CH_D)])

        @pl.when(cid == 1)
        def _():
            blocks(x1h, MUL, MUL)
            plsc.subcore_barrier()
            pltpu.sync_copy(acc.at[pl.ds(r0, CH_D)],
                            out1h.at[pl.ds(r0, CH_D)])

    return sc_kernel


def kernel(x, edge_attr, edge_weight, edge_dst, edge_src):
    N, F = x.shape
    E0 = edge_src.shape[0]
    K = -(-E0 // (NT * B))
    K = K + (K % 2)
    Ep = NT * K * B
    Npad = -(-N // 128) * 128

    perm0 = np.concatenate([np.arange(MUL), MUL + 3 * np.arange(MUL)])
    perm1 = np.concatenate([MUL + 1 + 3 * np.arange(MUL),
                            MUL + 2 + 3 * np.arange(MUL)])
    x = x.astype(jnp.float32)
    x0p = jnp.take(x, perm0, axis=1)
    x1p = jnp.take(x, perm1, axis=1)

    pad = Ep - E0
    assert E0 % CHUNK == 0
    srcp = edge_src.astype(jnp.int32).reshape(E0 // CHUNK, CHUNK)
    dstp = jnp.concatenate(
        [edge_dst.astype(jnp.int32), jnp.full((pad,), N, jnp.int32)]
    ).reshape(Ep // CHUNK, CHUNK)
    attrp = jnp.concatenate(
        [edge_attr.reshape(E0).astype(jnp.float32),
         jnp.zeros((pad,), jnp.float32)])
    wq = edge_weight.astype(jnp.float32)

    sc = _make_sc_kernel(Npad, E0, Ep, K)
    op0, op1 = sc(x0p, x1p, attrp, wq, srcp, dstp)

    inv = np.argsort(np.concatenate([perm0, perm1]))
    full = jnp.concatenate([op0[:N], op1[:N]], axis=1)
    return jnp.take(full, inv, axis=1)


# speedup vs baseline: 3.5314x; 1.0007x over previous
import functools

import jax
import jax.numpy as jnp
import numpy as np
from jax import lax
from jax.experimental import pallas as pl
from jax.experimental.pallas import tpu as pltpu
from jax.experimental.pallas import tpu_sc as plsc

MUL = 16
LANES = 16
W2 = 2 * MUL
NT = 16
B = 256
CHUNK = 128
NCH = B // CHUNK
ZR = 32


def _make_sc_kernel(Npad, E0, Ep, K):
    CH_D = Npad // NT
    RBV = E0 // CHUNK - NCH
    WBV = E0 - B
    NZ = CH_D // ZR
    ZT = CH_D % ZR
    mesh = plsc.VectorSubcoreMesh(core_axis_name="c", subcore_axis_name="s")

    @functools.partial(
        pl.kernel, mesh=mesh,
        compiler_params=pltpu.CompilerParams(use_tc_tiling_on_sc=False),
        out_type=(jax.ShapeDtypeStruct((Npad, W2), jnp.float32),
                  jax.ShapeDtypeStruct((Npad, W2), jnp.float32)),
        scratch_types=[
            pltpu.VMEM((NCH, CHUNK), jnp.int32),
            pltpu.VMEM((NCH, CHUNK), jnp.int32),
            pltpu.VMEM((NCH, CHUNK), jnp.int32),
            pltpu.VMEM((B,), jnp.float32),
            pltpu.VMEM((B, W2), jnp.float32),
            pltpu.VMEM((B, W2), jnp.float32),
            pltpu.VMEM((B, W2), jnp.float32),
            pltpu.VMEM((ZR, W2), jnp.float32),
            pltpu.VMEM_SHARED((Npad, W2), jnp.float32),
            pltpu.SemaphoreType.DMA,
            pltpu.SemaphoreType.DMA,
            pltpu.SemaphoreType.DMA,
        ])
    def sc_kernel(x0h, x1h, attrh, wh, srch, dsth, out0h, out1h,
                  src_v, dst0_v, dst1_v, attr_v, w_v, xr0_v, xr1_v, zero_v,
                  acc, gsem, ssem0, ssem1):
        cid = lax.axis_index("c")
        sid = lax.axis_index("s")
        dst_v = (dst0_v, dst1_v)
        xr_v = (xr0_v, xr1_v)
        ssem = (ssem0, ssem1)

        def zrow(r, c):
            z = jnp.zeros((LANES,), jnp.float32)
            zero_v[r, pl.ds(0, LANES)] = z
            zero_v[r, pl.ds(LANES, LANES)] = z
            return c
        lax.fori_loop(0, ZR, zrow, 0)

        r0 = sid * CH_D
        for t in range(NZ):
            pltpu.sync_copy(zero_v, acc.at[pl.ds(r0 + t * ZR, ZR)])
        if ZT:
            pltpu.sync_copy(zero_v.at[pl.ds(0, ZT)],
                            acc.at[pl.ds(r0 + NZ * ZR, ZT)])
        plsc.subcore_barrier()

        def scat_waits(p):
            for j in range(NCH):
                pltpu.make_async_copy(
                    xr_v[p].at[pl.ds(j * CHUNK, CHUNK)],
                    acc.at[dst_v[p].at[j]], ssem[p]).wait()

        def blocks(xh, wa, wb):
            def phase(i, p, blk):
                ebase = blk * B
                rb = blk * NCH

                @pl.when(i > 0)
                def _():
                    scat_waits(p)

                rb_s = jnp.minimum(rb, RBV)
                web = pl.multiple_of(jnp.minimum(ebase, WBV), CHUNK)
                pltpu.sync_copy(srch.at[pl.ds(rb_s, NCH)], src_v)
                pltpu.sync_copy(dsth.at[pl.ds(rb_s, NCH)], dst_v[p])
                pltpu.sync_copy(attrh.at[pl.ds(web, B)], attr_v)
                pltpu.sync_copy(wh.at[pl.ds(web, B)], w_v)

                @pl.when(ebase > WBV)
                def _():
                    def az(r, c):
                        attr_v[pl.ds(r * LANES, LANES)] = jnp.zeros(
                            (LANES,), jnp.float32)
                        return c
                    lax.fori_loop(0, B // LANES, az, 0)
                cps = [pltpu.async_copy(
                           xh.at[src_v.at[j]],
                           xr_v[p].at[pl.ds(j * CHUNK, CHUNK)], gsem)
                       for j in range(NCH)]
                for cp in cps:
                    cp.wait()

                @plsc.parallel_loop(0, B // LANES)
                def group(g):
                    s16 = attr_v[pl.ds(g * LANES, LANES)]
                    for j2 in range(LANES):
                        e = g * LANES + j2
                        sj = jnp.broadcast_to(s16[j2], (LANES,))
                        swa = sj * w_v[e, pl.ds(wa, MUL)]
                        if wb == wa:
                            swb = swa
                        else:
                            swb = sj * w_v[e, pl.ds(wb, MUL)]
                        xr_v[p][e, pl.ds(0, MUL)] = (
                            xr_v[p][e, pl.ds(0, MUL)] * swa)
                        xr_v[p][e, pl.ds(MUL, MUL)] = (
                            xr_v[p][e, pl.ds(MUL, MUL)] * swb)

                for j in range(NCH):
                    pltpu.async_copy(xr_v[p].at[pl.ds(j * CHUNK, CHUNK)],
                                     acc.at[dst_v[p].at[j]], ssem[p],
                                     add=True)

            def pair(i, c):
                blk = sid * K + 2 * i
                phase(i, 0, blk)
                phase(i, 1, blk + 1)
                return c
            lax.fori_loop(0, K // 2, pair, 0)
            scat_waits(0)
            scat_waits(1)

        @pl.when(cid == 0)
        def _():
            blocks(x0h, 0, MUL)
            plsc.subcore_barrier()
            pltpu.sync_copy(acc.at[pl.ds(r0, CH_D)],
                            out0h.at[pl.ds(r0, CH_D)])

        @pl.when(cid == 1)
        def _():
            blocks(x1h, MUL, MUL)
            plsc.subcore_barrier()
            pltpu.sync_copy(acc.at[pl.ds(r0, CH_D)],
                            out1h.at[pl.ds(r0, CH_D)])

    return sc_kernel


def kernel(x, edge_attr, edge_weight, edge_dst, edge_src):
    N, F = x.shape
    E0 = edge_src.shape[0]
    K = -(-E0 // (NT * B))
    K = K + (K % 2)
    Ep = NT * K * B
    Npad = -(-N // 128) * 128

    perm0 = np.concatenate([np.arange(MUL), MUL + 3 * np.arange(MUL)])
    perm1 = np.concatenate([MUL + 1 + 3 * np.arange(MUL),
                            MUL + 2 + 3 * np.arange(MUL)])
    x = x.astype(jnp.float32)
    x0p = jnp.take(x, perm0, axis=1)
    x1p = jnp.take(x, perm1, axis=1)

    assert E0 % B == 0
    srcp = edge_src.astype(jnp.int32).reshape(E0 // CHUNK, CHUNK)
    dstp = edge_dst.astype(jnp.int32).reshape(E0 // CHUNK, CHUNK)
    attrp = edge_attr.reshape(E0).astype(jnp.float32)
    wq = edge_weight.astype(jnp.float32)

    sc = _make_sc_kernel(Npad, E0, Ep, K)
    op0, op1 = sc(x0p, x1p, attrp, wq, srcp, dstp)

    inv = np.argsort(np.concatenate([perm0, perm1]))
    full = jnp.concatenate([op0[:N], op1[:N]], axis=1)
    return jnp.take(full, inv, axis=1)


# speedup vs baseline: 3.6860x; 1.0438x over previous
import functools

import jax
import jax.numpy as jnp
import numpy as np
from jax import lax
from jax.experimental import pallas as pl
from jax.experimental.pallas import tpu as pltpu
from jax.experimental.pallas import tpu_sc as plsc

MUL = 16
LANES = 16
W2 = 2 * MUL
NT = 16
B = 256
CHUNK = 128
NCH = B // CHUNK
ZR = 32


def _make_sc_kernel(Npad, E0, Ep, K):
    CH_D = Npad // NT
    RBV = E0 // CHUNK - NCH
    WBV = E0 - B
    NZ = CH_D // ZR
    ZT = CH_D % ZR
    mesh = plsc.VectorSubcoreMesh(core_axis_name="c", subcore_axis_name="s")

    @functools.partial(
        pl.kernel, mesh=mesh,
        compiler_params=pltpu.CompilerParams(use_tc_tiling_on_sc=False),
        out_type=jax.ShapeDtypeStruct((Npad, 2 * W2), jnp.float32),
        scratch_types=[
            pltpu.VMEM((NCH, CHUNK), jnp.int32),
            pltpu.VMEM((NCH, CHUNK), jnp.int32),
            pltpu.VMEM((NCH, CHUNK), jnp.int32),
            pltpu.VMEM((B,), jnp.float32),
            pltpu.VMEM((B, W2), jnp.float32),
            pltpu.VMEM((B, W2), jnp.float32),
            pltpu.VMEM((B, W2), jnp.float32),
            pltpu.VMEM((ZR, W2), jnp.float32),
            pltpu.VMEM_SHARED((Npad, W2), jnp.float32),
            pltpu.SemaphoreType.DMA,
            pltpu.SemaphoreType.DMA,
            pltpu.SemaphoreType.DMA,
        ])
    def sc_kernel(x0h, x1h, attrh, wh, srch, dsth, outh,
                  src_v, dst0_v, dst1_v, attr_v, w_v, xr0_v, xr1_v, zero_v,
                  acc, gsem, ssem0, ssem1):
        cid = lax.axis_index("c")
        sid = lax.axis_index("s")
        dst_v = (dst0_v, dst1_v)
        xr_v = (xr0_v, xr1_v)
        ssem = (ssem0, ssem1)

        def zrow(r, c):
            z = jnp.zeros((LANES,), jnp.float32)
            zero_v[r, pl.ds(0, LANES)] = z
            zero_v[r, pl.ds(LANES, LANES)] = z
            return c
        lax.fori_loop(0, ZR, zrow, 0)

        r0 = sid * CH_D
        for t in range(NZ):
            pltpu.sync_copy(zero_v, acc.at[pl.ds(r0 + t * ZR, ZR)])
        if ZT:
            pltpu.sync_copy(zero_v.at[pl.ds(0, ZT)],
                            acc.at[pl.ds(r0 + NZ * ZR, ZT)])
        plsc.subcore_barrier()

        def scat_waits(p):
            for j in range(NCH):
                pltpu.make_async_copy(
                    xr_v[p].at[pl.ds(j * CHUNK, CHUNK)],
                    acc.at[dst_v[p].at[j]], ssem[p]).wait()

        def blocks(xh, wa, wb):
            def phase(i, p, blk):
                ebase = blk * B
                rb = blk * NCH

                @pl.when(i > 0)
                def _():
                    scat_waits(p)

                rb_s = jnp.minimum(rb, RBV)
                web = pl.multiple_of(jnp.minimum(ebase, WBV), CHUNK)
                pltpu.sync_copy(srch.at[pl.ds(rb_s, NCH)], src_v)
                pltpu.sync_copy(dsth.at[pl.ds(rb_s, NCH)], dst_v[p])
                pltpu.sync_copy(attrh.at[pl.ds(web, B)], attr_v)
                pltpu.sync_copy(wh.at[pl.ds(web, B)], w_v)

                @pl.when(ebase > WBV)
                def _():
                    def az(r, c):
                        attr_v[pl.ds(r * LANES, LANES)] = jnp.zeros(
                            (LANES,), jnp.float32)
                        return c
                    lax.fori_loop(0, B // LANES, az, 0)
                cps = [pltpu.async_copy(
                           xh.at[src_v.at[j]],
                           xr_v[p].at[pl.ds(j * CHUNK, CHUNK)], gsem)
                       for j in range(NCH)]
                for cp in cps:
                    cp.wait()

                @plsc.parallel_loop(0, B // LANES)
                def group(g):
                    s16 = attr_v[pl.ds(g * LANES, LANES)]
                    for j2 in range(LANES):
                        e = g * LANES + j2
                        sj = jnp.broadcast_to(s16[j2], (LANES,))
                        swa = sj * w_v[e, pl.ds(wa, MUL)]
                        if wb == wa:
                            swb = swa
                        else:
                            swb = sj * w_v[e, pl.ds(wb, MUL)]
                        xr_v[p][e, pl.ds(0, MUL)] = (
                            xr_v[p][e, pl.ds(0, MUL)] * swa)
                        xr_v[p][e, pl.ds(MUL, MUL)] = (
                            xr_v[p][e, pl.ds(MUL, MUL)] * swb)

                for j in range(NCH):
                    pltpu.async_copy(xr_v[p].at[pl.ds(j * CHUNK, CHUNK)],
                                     acc.at[dst_v[p].at[j]], ssem[p],
                                     add=True)

            def pair(i, c):
                blk = sid * K + 2 * i
                phase(i, 0, blk)
                phase(i, 1, blk + 1)
                return c
            lax.fori_loop(0, K // 2, pair, 0)
            scat_waits(0)
            scat_waits(1)

        @pl.when(cid == 0)
        def _():
            blocks(x0h, 0, MUL)
            plsc.subcore_barrier()
            pltpu.sync_copy(acc.at[pl.ds(r0, CH_D)],
                            outh.at[pl.ds(r0, CH_D), pl.ds(0, W2)])

        @pl.when(cid == 1)
        def _():
            blocks(x1h, MUL, MUL)
            plsc.subcore_barrier()
            pltpu.sync_copy(acc.at[pl.ds(r0, CH_D)],
                            outh.at[pl.ds(r0, CH_D), pl.ds(W2, W2)])

    return sc_kernel


def kernel(x, edge_attr, edge_weight, edge_dst, edge_src):
    N, F = x.shape
    E0 = edge_src.shape[0]
    K = -(-E0 // (NT * B))
    K = K + (K % 2)
    Ep = NT * K * B
    Npad = -(-N // 128) * 128

    perm0 = np.concatenate([np.arange(MUL), MUL + 3 * np.arange(MUL)])
    perm1 = np.concatenate([MUL + 1 + 3 * np.arange(MUL),
                            MUL + 2 + 3 * np.arange(MUL)])
    x = x.astype(jnp.float32)
    x0p = jnp.take(x, perm0, axis=1)
    x1p = jnp.take(x, perm1, axis=1)

    assert E0 % B == 0
    srcp = edge_src.astype(jnp.int32).reshape(E0 // CHUNK, CHUNK)
    dstp = edge_dst.astype(jnp.int32).reshape(E0 // CHUNK, CHUNK)
    attrp = edge_attr.reshape(E0).astype(jnp.float32)
    wq = edge_weight.astype(jnp.float32)

    sc = _make_sc_kernel(Npad, E0, Ep, K)
    op = sc(x0p, x1p, attrp, wq, srcp, dstp)

    inv = np.argsort(np.concatenate([perm0, perm1]))
    return jnp.take(op[:N], inv, axis=1)
